# baseline (speedup 1.0000x reference)
import functools

import jax
import jax.numpy as jnp
from jax import lax
from jax.experimental import pallas as pl
from jax.experimental.pallas import tpu as pltpu
from jax.experimental.pallas import tpu_sc as plsc

B = 16384
N1 = 6
N2 = 146
N2P = 152
EMB = 64
EMBED = 128
OUT = 128
BLK = 2048
NB = B // BLK

NC = 2
NS = 16
NW = NC * NS
BPW = B // NW


def _argmax_first(x, n):
    blk = x.shape[0]
    iota = lax.broadcasted_iota(jnp.int32, (blk, n), 1)
    m = jnp.max(x, axis=1, keepdims=True)
    return jnp.min(jnp.where(x == m, iota, n), axis=1)


def _tc_body(cts_ref, smlss_ref, emb1_ref, emb2_ref, w1_ref, b1_ref, w2_ref,
             b2_ref, idx_ref, tab_ref):
    i1 = _argmax_first(cts_ref[...], N1)
    i2 = _argmax_first(smlss_ref[...], N2)
    idx_ref[...] = (i1 * N2P + i2).reshape(1, 1, BLK)

    @pl.when(pl.program_id(0) == 0)
    def _build_table():
        t1 = jnp.dot(emb1_ref[...], w1_ref[:EMB, :],
                     preferred_element_type=jnp.float32)
        t2 = jnp.dot(emb2_ref[...], w1_ref[EMB:, :],
                     preferred_element_type=jnp.float32)
        for i in range(N1):
            h = jnp.maximum(t2 + t1[i:i + 1, :] + b1_ref[...], 0.0)
            tab_ref[i * N2P:(i + 1) * N2P, :] = (
                jnp.dot(h, w2_ref[...], preferred_element_type=jnp.float32)
                + b2_ref[...])


@jax.jit
def _tc_stage(cts, smlss, emb1, emb2p, W1, b1, W2, b2):
    return pl.pallas_call(
        _tc_body,
        grid=(NB,),
        in_specs=[
            pl.BlockSpec((BLK, N1), lambda i: (i, 0)),
            pl.BlockSpec((BLK, N2), lambda i: (i, 0)),
            pl.BlockSpec((N1, EMB), lambda i: (0, 0)),
            pl.BlockSpec((N2P, EMB), lambda i: (0, 0)),
            pl.BlockSpec((EMBED, EMBED), lambda i: (0, 0)),
            pl.BlockSpec((1, EMBED), lambda i: (0, 0)),
            pl.BlockSpec((EMBED, OUT), lambda i: (0, 0)),
            pl.BlockSpec((1, OUT), lambda i: (0, 0)),
        ],
        out_specs=[
            pl.BlockSpec((1, 1, BLK), lambda i: (i, 0, 0)),
            pl.BlockSpec((N1 * N2P, OUT), lambda i: (0, 0)),
        ],
        out_shape=[
            jax.ShapeDtypeStruct((NB, 1, BLK), jnp.int32),
            jax.ShapeDtypeStruct((N1 * N2P, OUT), jnp.float32),
        ],
    )(cts, smlss, emb1, emb2p, W1, b1.reshape(1, EMBED), W2,
      b2.reshape(1, OUT))


_sc_mesh = plsc.VectorSubcoreMesh(core_axis_name="c", subcore_axis_name="s")


@jax.jit
@functools.partial(
    pl.kernel, mesh=_sc_mesh,
    out_type=jax.ShapeDtypeStruct((B, OUT), jnp.float32),
    scratch_types=[
        pltpu.VMEM((BPW,), jnp.int32),
        pltpu.VMEM((BPW, OUT), jnp.float32),
        pltpu.SemaphoreType.DMA,
    ],
)
def _sc_gather(tab_hbm, idx_hbm, out_hbm, idx_v, rows_v, sem):
    wid = lax.axis_index("s") * NC + lax.axis_index("c")
    base = wid * BPW
    pltpu.sync_copy(idx_hbm.at[pl.ds(base, BPW)], idx_v)
    pltpu.async_copy(tab_hbm.at[idx_v], rows_v, sem).wait()
    pltpu.sync_copy(rows_v, out_hbm.at[pl.ds(base, BPW)])


def kernel(cts, smlss, emb1, emb2, W1, b1, W2, b2):
    emb2p = jnp.concatenate(
        [emb2, jnp.zeros((N2P - N2, EMB), jnp.float32)], axis=0)
    idx3, tab = _tc_stage(cts, smlss, emb1, emb2p, W1, b1, W2, b2)
    return _sc_gather(tab, idx3.reshape(B))


# speedup vs baseline: 1.1149x; 1.1149x over previous
import functools

import jax
import jax.numpy as jnp
import numpy as np
from jax import lax
from jax.experimental import pallas as pl
from jax.experimental.pallas import tpu as pltpu
from jax.experimental.pallas import tpu_sc as plsc

B = 16384
N1 = 6
N2 = 146
N2P = 152
EMB = 64
EMBED = 128
OUT = 128
BLK = 2048
NB = B // BLK

NC = 2
NS = 16
NW = NC * NS
BPW = B // NW


def _argmax_first_f(x, iota_row, n):
    blk = x.shape[0]
    iota = jnp.broadcast_to(iota_row, (blk, n))
    m = jnp.max(x, axis=1, keepdims=True)
    return jnp.min(jnp.where(x == m, iota, float(n)), axis=1)


def _tc_body(cts_ref, smlss_ref, emb1_ref, emb2_ref, w1_ref, b1_ref, w2_ref,
             b2_ref, io1_ref, io2_ref, idx_ref, tab_ref):
    io1 = io1_ref[...]
    io2 = io2_ref[...]
    i1 = _argmax_first_f(cts_ref[...], io1, N1)
    i2 = _argmax_first_f(smlss_ref[...], io2, N2)
    oh1 = (jnp.broadcast_to(io1, (BLK, N1)) == i1[:, None]).astype(jnp.float32)
    oh2 = (jnp.broadcast_to(io2, (BLK, N2)) == i2[:, None]).astype(jnp.float32)
    lane1 = lax.dot_general(io1 * float(N2P), oh1, (((1,), (1,)), ((), ())),
                            preferred_element_type=jnp.float32)
    lane2 = lax.dot_general(io2, oh2, (((1,), (1,)), ((), ())),
                            preferred_element_type=jnp.float32)
    idx_ref[...] = (lane1 + lane2 + 0.5).reshape(1, 1, BLK).astype(jnp.int32)

    @pl.when(pl.program_id(0) == 0)
    def _build_table():
        t1 = jnp.dot(emb1_ref[...], w1_ref[:EMB, :],
                     preferred_element_type=jnp.float32)
        t2 = jnp.dot(emb2_ref[...], w1_ref[EMB:, :],
                     preferred_element_type=jnp.float32)
        for i in range(N1):
            h = jnp.maximum(t2 + t1[i:i + 1, :] + b1_ref[...], 0.0)
            tab_ref[i * N2P:(i + 1) * N2P, :] = (
                jnp.dot(h, w2_ref[...], preferred_element_type=jnp.float32)
                + b2_ref[...])


@jax.jit
def _tc_stage(cts, smlss, emb1, emb2p, W1, b1, W2, b2):
    return pl.pallas_call(
        _tc_body,
        grid=(NB,),
        in_specs=[
            pl.BlockSpec((BLK, N1), lambda i: (i, 0)),
            pl.BlockSpec((BLK, N2), lambda i: (i, 0)),
            pl.BlockSpec((N1, EMB), lambda i: (0, 0)),
            pl.BlockSpec((N2P, EMB), lambda i: (0, 0)),
            pl.BlockSpec((EMBED, EMBED), lambda i: (0, 0)),
            pl.BlockSpec((1, EMBED), lambda i: (0, 0)),
            pl.BlockSpec((EMBED, OUT), lambda i: (0, 0)),
            pl.BlockSpec((1, OUT), lambda i: (0, 0)),
            pl.BlockSpec((1, N1), lambda i: (0, 0)),
            pl.BlockSpec((1, N2), lambda i: (0, 0)),
        ],
        out_specs=[
            pl.BlockSpec((1, 1, BLK), lambda i: (i, 0, 0)),
            pl.BlockSpec((N1 * N2P, OUT), lambda i: (0, 0)),
        ],
        out_shape=[
            jax.ShapeDtypeStruct((NB, 1, BLK), jnp.int32),
            jax.ShapeDtypeStruct((N1 * N2P, OUT), jnp.float32),
        ],
    )(cts, smlss, emb1, emb2p, W1, b1.reshape(1, EMBED), W2,
      b2.reshape(1, OUT),
      jnp.asarray(np.arange(N1, dtype=np.float32)).reshape(1, N1),
      jnp.asarray(np.arange(N2, dtype=np.float32)).reshape(1, N2))


_sc_mesh = plsc.VectorSubcoreMesh(core_axis_name="c", subcore_axis_name="s")


@jax.jit
@functools.partial(
    pl.kernel, mesh=_sc_mesh,
    out_type=jax.ShapeDtypeStruct((B, OUT), jnp.float32),
    scratch_types=[
        pltpu.VMEM((BPW,), jnp.int32),
        pltpu.VMEM((BPW, OUT), jnp.float32),
        pltpu.SemaphoreType.DMA,
    ],
)
def _sc_gather(tab_hbm, idx_hbm, out_hbm, idx_v, rows_v, sem):
    wid = lax.axis_index("s") * NC + lax.axis_index("c")
    base = wid * BPW
    pltpu.sync_copy(idx_hbm.at[pl.ds(base, BPW)], idx_v)
    pltpu.async_copy(tab_hbm.at[idx_v], rows_v, sem).wait()
    pltpu.sync_copy(rows_v, out_hbm.at[pl.ds(base, BPW)])


def kernel(cts, smlss, emb1, emb2, W1, b1, W2, b2):
    emb2p = jnp.concatenate(
        [emb2, jnp.zeros((N2P - N2, EMB), jnp.float32)], axis=0)
    idx3, tab = _tc_stage(cts, smlss, emb1, emb2p, W1, b1, W2, b2)
    return _sc_gather(tab, idx3.reshape(B))
